# baseline (speedup 1.0000x reference)
import functools

import jax
import jax.numpy as jnp
from jax import lax
from jax.experimental import pallas as pl
from jax.experimental.pallas import tpu as pltpu
from jax.experimental.pallas import tpu_sc as plsc

DIM = 64
BATCH = 16384
HIST = 50

B = BATCH * HIST
NW = 32
CHUNK = 128
N_CHUNKS = B // CHUNK
C_PER_W = N_CHUNKS // NW
G_PER_W = C_PER_W // 2
N_QUADS = G_PER_W // 4
TPG = BATCH // (2 * CHUNK)


def _make_gather():
    mesh = plsc.VectorSubcoreMesh(core_axis_name="c", subcore_axis_name="s")

    @functools.partial(
        pl.kernel,
        mesh=mesh,
        out_type=jax.ShapeDtypeStruct((HIST, DIM // 8, TPG, 2048),
                                      jnp.float32),
        scratch_types=[
            pltpu.VMEM((C_PER_W, CHUNK), jnp.int32),
            [pltpu.VMEM((CHUNK, DIM), jnp.float32) for _ in range(8)],
            pltpu.VMEM((8 * 2048,), jnp.float32),
            [pltpu.SemaphoreType.DMA for _ in range(8)],
            pltpu.SemaphoreType.DMA,
        ],
        compiler_params=pltpu.CompilerParams(use_tc_tiling_on_sc=False,
                                             needs_layout_passes=False),
    )
    def gather_kernel(table_hbm, idx_hbm, out_hbm, idx_v, rows, tbuf,
                      gsem, wsem):
        wid = lax.axis_index("s") * 2 + lax.axis_index("c")
        chunk0 = wid * C_PER_W
        group0 = wid * G_PER_W
        pltpu.sync_copy(idx_hbm.at[pl.ds(chunk0, C_PER_W)], idx_v)

        lane = lax.broadcasted_iota(jnp.int32, (16,), 0)
        jmods = [(lane + r) % 16 for r in range(16)]
        jm7l = [(jmods[r] // 8) * 2048 + (jmods[r] % 8) * CHUNK + lane
                for r in range(16)]

        def fire(c, slot):
            pltpu.async_copy(table_hbm.at[idx_v.at[c]], rows[slot],
                             gsem[slot])

        def drain(c, slot):
            pltpu.make_async_copy(table_hbm.at[idx_v.at[c]], rows[slot],
                                  gsem[slot]).wait()

        def wait_out(h, t):
            for tr in range(8):
                pltpu.make_async_copy(tbuf.at[pl.ds(tr * 2048, 2048)],
                                      out_hbm.at[h, tr, t], wsem).wait()

        for s8 in range(8):
            fire(s8, s8)

        def quad_body(g4, _):
            for u in range(4):
                g = 4 * g4 + u
                gg = group0 + g
                h = gg // TPG
                t = gg % TPG
                drain(2 * g, 2 * u)
                drain(2 * g + 1, 2 * u + 1)

                if u > 0:
                    wait_out(h, t)
                else:
                    @pl.when(g4 >= 1)
                    def _():
                        wait_out(h, t)

                for cc in range(2):
                    sl = 2 * u + cc

                    def bloop(bg, _, sl=sl, cc=cc):
                        b0 = bg * 16
                        bvec = b0 + lane
                        for d0 in range(0, DIM, 16):
                            sbase = (d0 // 8) * 2048 + cc * 1024 + b0
                            for r in range(16):
                                v = plsc.load_gather(
                                    rows[sl], [bvec, jmods[r] + d0])
                                plsc.store_scatter(
                                    tbuf, [jm7l[r] + sbase], v)
                        return 0

                    lax.fori_loop(0, CHUNK // 16, bloop, 0)

                @pl.when(g4 < N_QUADS - 1)
                def _():
                    fire(2 * (g + 4), 2 * u)
                    fire(2 * (g + 4) + 1, 2 * u + 1)

                for tr in range(8):
                    pltpu.async_copy(tbuf.at[pl.ds(tr * 2048, 2048)],
                                     out_hbm.at[h, tr, t], wsem)
            return 0

        lax.fori_loop(0, N_QUADS, quad_body, 0)

        gl = group0 + G_PER_W - 1
        wait_out(gl // TPG, gl % TPG)

    return gather_kernel


_gather = _make_gather()


def kernel(data, ivectors, ovectors):
    idx = data.T.reshape(N_CHUNKS, CHUNK).astype(jnp.int32)
    o5 = _gather(ivectors, idx)
    out = (o5.reshape(HIST, DIM // 8, TPG, 2, 8, CHUNK)
           .transpose(2, 3, 5, 0, 1, 4)
           .reshape(BATCH, HIST, DIM))
    return out


# speedup vs baseline: 1.0263x; 1.0263x over previous
import functools

import jax
import jax.numpy as jnp
from jax import lax
from jax.experimental import pallas as pl
from jax.experimental.pallas import tpu as pltpu
from jax.experimental.pallas import tpu_sc as plsc

DIM = 64
BATCH = 16384
HIST = 50

B = BATCH * HIST
NW = 32
CHUNK = 128
N_CHUNKS = B // CHUNK
C_PER_W = N_CHUNKS // NW
N_QUADS = C_PER_W // 4
BT = BATCH // CHUNK


def _make_gather():
    mesh = plsc.VectorSubcoreMesh(core_axis_name="c", subcore_axis_name="s")

    @functools.partial(
        pl.kernel,
        mesh=mesh,
        out_type=jax.ShapeDtypeStruct((HIST, DIM // 8, BT, 1024),
                                      jnp.float32),
        scratch_types=[
            pltpu.VMEM((C_PER_W, CHUNK), jnp.int32),
            [pltpu.VMEM((CHUNK, DIM), jnp.float32) for _ in range(4)],
            [pltpu.VMEM((8 * 1024,), jnp.float32) for _ in range(2)],
            [pltpu.SemaphoreType.DMA for _ in range(4)],
            [pltpu.SemaphoreType.DMA for _ in range(2)],
        ],
        compiler_params=pltpu.CompilerParams(use_tc_tiling_on_sc=False,
                                             needs_layout_passes=False),
    )
    def gather_kernel(table_hbm, idx_hbm, out_hbm, idx_v, rows, tbuf,
                      gsem, wsem):
        wid = lax.axis_index("s") * 2 + lax.axis_index("c")
        chunk0 = wid * C_PER_W
        pltpu.sync_copy(idx_hbm.at[pl.ds(chunk0, C_PER_W)], idx_v)

        lane = lax.broadcasted_iota(jnp.int32, (16,), 0)
        jmods = [(lane + r) % 16 for r in range(16)]

        def fire(c, slot):
            pltpu.async_copy(table_hbm.at[idx_v.at[c]], rows[slot],
                             gsem[slot])

        def drain(c, slot):
            pltpu.make_async_copy(table_hbm.at[idx_v.at[c]], rows[slot],
                                  gsem[slot]).wait()

        def wait_out(h, tc, p):
            for tr in range(8):
                pltpu.make_async_copy(tbuf[p].at[pl.ds(tr * 1024, 1024)],
                                      out_hbm.at[h, tr, tc],
                                      wsem[p]).wait()

        fire(0, 0)
        fire(1, 1)

        def quad_body(q, _):
            for u in range(4):
                c = 4 * q + u
                p = u & 1
                cg = chunk0 + c
                h = cg // BT
                tc = cg % BT
                drain(c, u)

                @pl.when(c + 2 < C_PER_W)
                def _():
                    fire(c + 2, (u + 2) & 3)

                if u >= 2:
                    wait_out(h, tc, p)
                else:
                    @pl.when(q >= 1)
                    def _():
                        wait_out(h, tc, p)

                def bloop(bg, _, u=u, p=p):
                    b0 = bg * 16
                    bvec = b0 + lane
                    for d0 in range(0, DIM, 16):
                        dbase = d0 * CHUNK + b0 + lane
                        for r in range(16):
                            v = plsc.load_gather(
                                rows[u], [bvec, jmods[r] + d0])
                            plsc.store_scatter(
                                tbuf[p], [(jmods[r] * CHUNK) + dbase], v)
                    return 0

                lax.fori_loop(0, CHUNK // 16, bloop, 0)

                for tr in range(8):
                    pltpu.async_copy(tbuf[p].at[pl.ds(tr * 1024, 1024)],
                                     out_hbm.at[h, tr, tc], wsem[p])
            return 0

        lax.fori_loop(0, N_QUADS, quad_body, 0)

        for c in (C_PER_W - 2, C_PER_W - 1):
            cg = chunk0 + c
            wait_out(cg // BT, cg % BT, c & 1)

    return gather_kernel


_gather = _make_gather()


def kernel(data, ivectors, ovectors):
    idx = data.T.reshape(N_CHUNKS, CHUNK).astype(jnp.int32)
    o5 = _gather(ivectors, idx)
    out = (o5.reshape(HIST, DIM // 8, BT, 8, CHUNK)
           .transpose(2, 4, 0, 1, 3)
           .reshape(BATCH, HIST, DIM))
    return out


# speedup vs baseline: 1.1126x; 1.0840x over previous
import functools

import jax
import jax.numpy as jnp
from jax import lax
from jax.experimental import pallas as pl
from jax.experimental.pallas import tpu as pltpu
from jax.experimental.pallas import tpu_sc as plsc

DIM = 64
BATCH = 16384
HIST = 50

B = BATCH * HIST
NW = 32
CHUNK = 128
N_CHUNKS = B // CHUNK
C_PER_W = N_CHUNKS // NW
N_PAIRS = C_PER_W // 2
BT = BATCH // CHUNK


def _make_gather():
    mesh = plsc.VectorSubcoreMesh(core_axis_name="c", subcore_axis_name="s")

    @functools.partial(
        pl.kernel,
        mesh=mesh,
        out_type=jax.ShapeDtypeStruct((HIST, DIM // 8, BT, 1024),
                                      jnp.float32),
        scratch_types=[
            pltpu.VMEM((C_PER_W, CHUNK), jnp.int32),
            [pltpu.VMEM((CHUNK, DIM), jnp.float32) for _ in range(2)],
            [pltpu.VMEM((8 * 1024,), jnp.float32) for _ in range(2)],
            [pltpu.SemaphoreType.DMA for _ in range(2)],
            [pltpu.SemaphoreType.DMA for _ in range(2)],
        ],
        compiler_params=pltpu.CompilerParams(use_tc_tiling_on_sc=False,
                                             needs_layout_passes=False),
    )
    def gather_kernel(table_hbm, idx_hbm, out_hbm, idx_v, rows, tbuf,
                      gsem, wsem):
        wid = lax.axis_index("s") * 2 + lax.axis_index("c")
        chunk0 = wid * C_PER_W
        pltpu.sync_copy(idx_hbm.at[pl.ds(chunk0, C_PER_W)], idx_v)

        lane = lax.broadcasted_iota(jnp.int32, (16,), 0)
        rotidx = (lane + 1) % 16

        def fire(c, p):
            pltpu.async_copy(table_hbm.at[idx_v.at[c]], rows[p], gsem[p])

        def drain(c, p):
            pltpu.make_async_copy(table_hbm.at[idx_v.at[c]], rows[p],
                                  gsem[p]).wait()

        def wait_out(h, tc, p):
            for tr in range(8):
                pltpu.make_async_copy(tbuf[p].at[pl.ds(tr * 1024, 1024)],
                                      out_hbm.at[h, tr, tc],
                                      wsem[p]).wait()

        fire(0, 0)

        def pair_body(c2, _):
            for p in range(2):
                c = 2 * c2 + p
                cg = chunk0 + c
                h = cg // BT
                tc = cg % BT
                drain(c, p)

                @pl.when(c + 1 < C_PER_W)
                def _():
                    fire(c + 1, 1 - p)

                @pl.when(c2 >= 1)
                def _():
                    wait_out(h, tc, p)

                def bloop(bg, _, p=p):
                    b0 = bg * 16
                    bvec = b0 + lane
                    for d0 in range(0, DIM, 16):
                        sb2 = bvec + d0 * CHUNK
                        jm = lane
                        for r in range(16):
                            v = plsc.load_gather(
                                rows[p], [bvec, jm + d0])
                            plsc.store_scatter(
                                tbuf[p], [(jm * CHUNK) + sb2], v)
                            if r < 15:
                                jm = jm[rotidx]
                    return 0

                lax.fori_loop(0, CHUNK // 16, bloop, 0)

                for tr in range(8):
                    pltpu.async_copy(tbuf[p].at[pl.ds(tr * 1024, 1024)],
                                     out_hbm.at[h, tr, tc], wsem[p])
            return 0

        lax.fori_loop(0, N_PAIRS, pair_body, 0)

        for c in (C_PER_W - 2, C_PER_W - 1):
            cg = chunk0 + c
            wait_out(cg // BT, cg % BT, c & 1)

    return gather_kernel


_gather = _make_gather()


def kernel(data, ivectors, ovectors):
    idx = data.T.reshape(N_CHUNKS, CHUNK).astype(jnp.int32)
    o5 = _gather(ivectors, idx)
    out = (o5.reshape(HIST, DIM // 8, BT, 8, CHUNK)
           .transpose(2, 4, 0, 1, 3)
           .reshape(BATCH, HIST, DIM))
    return out
